# Initial kernel scaffold; baseline (speedup 1.0000x reference)
import jax
import jax.numpy as jnp
from jax.experimental import pallas as pl


def kernel(params, action_mask, time_step, progress, max_env_len, state_depot_hoop, have_raw_hoops, state_depot_bending_tube, have_raw_bending_tube, station_state_inner_left, station_state_inner_right, station_state_outer_left, station_state_outer_right, cutting_machine_state, is_full_products, produce_product_req, raw_products, worker_state, worker_task, worker_pose, worker_fatigue_phy, worker_fatigue_psy, phy_fatigue_coe, agv_state, agv_task, agv_pose, box_state, box_task, box_pose, token_mask):
    raise NotImplementedError("write your pallas kernel here")


# speedup vs baseline: 2.4959x; 2.4959x over previous
import functools
import math

import jax
import jax.numpy as jnp
from jax.experimental import pallas as pl

H = 128
NT = 79
SQ = math.sqrt(float(H))

_TABLES = ('depot_hoop', 'have_raw_hoops', 'depot_bt', 'have_raw_bt',
           'inner_left', 'inner_right', 'outer_left', 'outer_right',
           'cutting', 'is_full', 'produce_req', 'raw_product',
           'worker_state', 'worker_task', 'worker_pose',
           'agv_state', 'agv_task', 'agv_pose', 'box_state', 'box_task')
_SMLPS = ('max_env_len_mlp', 'time_step_mlp', 'progress_mlp',
          'phy_fat_mlp', 'psy_fat_mlp', 'coe_mlp')


def _tc_body(x_ref, w1a_ref, w2a_ref, w1r_ref, w2s_ref, tab_ref,
             gidx_ref, xs_ref, mask_ref, out_ref):
    h = jnp.maximum(jnp.dot(x_ref[...], w1a_ref[...],
                            preferred_element_type=jnp.float32), 0.0)
    am = jnp.dot(h, w2a_ref[...], preferred_element_type=jnp.float32)

    w1r = w1r_ref[...]
    vp6 = jax.lax.dot_general(jnp.maximum(w1r, 0.0)[:, None, :], w2s_ref[...],
                              (((2,), (1,)), ((0,), (0,))),
                              preferred_element_type=jnp.float32)[:, 0, :]
    vn6 = jax.lax.dot_general(jnp.minimum(w1r, 0.0)[:, None, :], w2s_ref[...],
                              (((2,), (1,)), ((0,), (0,))),
                              preferred_element_type=jnp.float32)[:, 0, :]
    vp39 = jnp.concatenate([vp6[0:3],
                            jnp.broadcast_to(vp6[3:4], (3, H)),
                            jnp.broadcast_to(vp6[4:5], (3, H)),
                            jnp.broadcast_to(vp6[5:6], (30, H))], axis=0)
    vn39 = jnp.concatenate([vn6[0:3],
                            jnp.broadcast_to(vn6[3:4], (3, H)),
                            jnp.broadcast_to(vn6[4:5], (3, H)),
                            jnp.broadcast_to(vn6[5:6], (30, H))], axis=0)
    xs = xs_ref[...]
    s = (jnp.maximum(xs, 0.0)[:, :, None] * vp39[None] +
         jnp.minimum(xs, 0.0)[:, :, None] * vn39[None])

    gidx = gidx_ref[...]
    onehot = (gidx[:, :, None] ==
              jax.lax.broadcasted_iota(jnp.int32, (1, 1, 123), 2)
              ).astype(jnp.float32)
    g = jax.lax.dot_general(onehot, tab_ref[...], (((2,), (0,)), ((), ())),
                            preferred_element_type=jnp.float32)

    m = mask_ref[...] * SQ
    out_ref[:, 0:1, :] = am[:, None, :] * m[:, 0:1, None]
    out_ref[:, 1:13, :] = g[:, 0:12] * m[:, 1:13, None]
    out_ref[:, 13:16, :] = s[:, 0:3] * m[:, 13:16, None]
    out_ref[:, 16:25, :] = g[:, 12:21] * m[:, 16:25, None]
    out_ref[:, 25:61, :] = s[:, 3:39] * m[:, 25:61, None]
    out_ref[:, 61:79, :] = g[:, 21:39] * m[:, 61:79, None]


@functools.partial(jax.jit, static_argnames=('interpret',))
def _run(x, w1a, w2a, w1r, w2s, tab, gidx, xs, mask, interpret=False):
    B = x.shape[0]
    BB = 256
    grid = (B // BB,)
    full = lambda shape: pl.BlockSpec(shape, lambda i: (0,) * len(shape))
    return pl.pallas_call(
        _tc_body,
        grid=grid,
        in_specs=[
            pl.BlockSpec((BB, 10), lambda i: (i, 0)),
            full((10, H)), full((H, H)), full((6, H)), full((6, H, H)),
            full((123, H)),
            pl.BlockSpec((BB, 39), lambda i: (i, 0)),
            pl.BlockSpec((BB, 39), lambda i: (i, 0)),
            pl.BlockSpec((BB, NT), lambda i: (i, 0)),
        ],
        out_specs=pl.BlockSpec((BB, NT, H), lambda i: (i, 0, 0)),
        out_shape=jax.ShapeDtypeStruct((B, NT, H), jnp.float32),
        interpret=interpret,
    )(x, w1a, w2a, w1r, w2s, tab, gidx, xs, mask)


def kernel(params, action_mask, time_step, progress, max_env_len,
           state_depot_hoop, have_raw_hoops, state_depot_bending_tube,
           have_raw_bending_tube, station_state_inner_left,
           station_state_inner_right, station_state_outer_left,
           station_state_outer_right, cutting_machine_state, is_full_products,
           produce_product_req, raw_products, worker_state, worker_task,
           worker_pose, worker_fatigue_phy, worker_fatigue_psy,
           phy_fatigue_coe, agv_state, agv_task, agv_pose, box_state,
           box_task, box_pose, token_mask, *, interpret=False):
    B = action_mask.shape[0]
    tab = jnp.concatenate([params[t] for t in _TABLES], axis=0)
    offs, o = {}, 0
    for t in _TABLES:
        offs[t] = o
        o += params[t].shape[0]

    singles = [(state_depot_hoop, 'depot_hoop'), (have_raw_hoops, 'have_raw_hoops'),
               (state_depot_bending_tube, 'depot_bt'), (have_raw_bending_tube, 'have_raw_bt'),
               (station_state_inner_left, 'inner_left'), (station_state_inner_right, 'inner_right'),
               (station_state_outer_left, 'outer_left'), (station_state_outer_right, 'outer_right'),
               (cutting_machine_state, 'cutting'), (is_full_products, 'is_full'),
               (produce_product_req, 'produce_req'), (raw_products, 'raw_product')]
    ne3 = [(worker_state, 'worker_state'), (worker_task, 'worker_task'),
           (worker_pose, 'worker_pose'), (agv_state, 'agv_state'),
           (agv_task, 'agv_task'), (agv_pose, 'agv_pose'),
           (box_state, 'box_state'), (box_task, 'box_task'),
           (box_pose, 'agv_pose')]
    gidx = jnp.concatenate([a[:, 0:1] + offs[t] for a, t in singles] +
                           [a[:, :, 0] + offs[t] for a, t in ne3], axis=1)

    xs = jnp.concatenate([max_env_len, time_step, progress,
                          worker_fatigue_phy[:, :, 0, 0],
                          worker_fatigue_psy[:, :, 0, 0],
                          phy_fatigue_coe.reshape(B, 30)], axis=1)

    w1r = jnp.stack([params[mp]['W1'][0] for mp in _SMLPS], axis=0)
    w2s = jnp.stack([params[mp]['W2'] for mp in _SMLPS], axis=0)
    amp = params['action_mask_mlp']
    return _run(action_mask, amp['W1'], amp['W2'], w1r, w2s, tab,
                gidx, xs, token_mask, interpret=interpret)
